# Initial kernel scaffold; baseline (speedup 1.0000x reference)
import jax
import jax.numpy as jnp
from jax.experimental import pallas as pl


def kernel(feature, proposal, image_shape, W1, b1, W2, b2, Wc, bc, Wr, br):
    raise NotImplementedError("write your pallas kernel here")


# speedup vs baseline: 4.8843x; 4.8843x over previous
import functools

import jax
import jax.numpy as jnp
import numpy as np
from jax.experimental import pallas as pl
from jax.experimental.pallas import tpu as pltpu

N_PROP = 1000
NP_PAD = 1024
C = 256
FH = 50
FW = 50
NUM_CLASSES = 21
POOL = 7
STRIDE = 16.0
HIDDEN = 1024
SCORE_THRESH = 0.05
NMS_THRESH = 0.5
NUM_DET = 100
MIN_SIZE = 1.0
REG_W = (10.0, 10.0, 5.0, 5.0)
BBOX_XFORM_CLIP = float(np.log(1000.0 / 16.0))

PADW = FW + 2
_PREC = jax.lax.Precision.DEFAULT
_INTERPRET = False


def _pool_kernel(ind_ref, lyx_ref, feat_ref, g_ref):
    s = pl.program_id(0)

    def roi_body(r, _):
        ly = lyx_ref[0, s, r]
        for j in range(POOL):
            lx = lyx_ref[1, j, r]
            base = ind_ref[s * POOL + j, r]
            w00 = (1.0 - ly) * (1.0 - lx)
            w01 = (1.0 - ly) * lx
            w10 = ly * (1.0 - lx)
            w11 = ly * lx
            v00 = feat_ref[base, :]
            v01 = feat_ref[base + 1, :]
            v10 = feat_ref[base + PADW, :]
            v11 = feat_ref[base + PADW + 1, :]
            g_ref[r, j * C:(j + 1) * C] = (v00 * w00 + v01 * w01 +
                                           v10 * w10 + v11 * w11)
        return 0

    jax.lax.fori_loop(0, NP_PAD, roi_body, 0, unroll=2)


def _run_pool(ind, lyx, feat_pad):
    kb = POOL * C
    return pl.pallas_call(
        _pool_kernel,
        grid=(POOL,),
        in_specs=[
            pl.BlockSpec(memory_space=pltpu.SMEM),
            pl.BlockSpec(memory_space=pltpu.SMEM),
            pl.BlockSpec((PADW * PADW, C), lambda s: (0, 0)),
        ],
        out_specs=pl.BlockSpec((NP_PAD, kb), lambda s: (0, s)),
        out_shape=jax.ShapeDtypeStruct((NP_PAD, POOL * POOL * C), jnp.float32),
        interpret=_INTERPRET,
    )(ind, lyx, feat_pad)


def _w1_kernel(p_ref, w1_ref, h1_ref):
    h1_ref[...] = jax.lax.dot_general(
        p_ref[...], w1_ref[...], (((1,), (0,)), ((), ())),
        preferred_element_type=jnp.float32)


def _run_w1(pooled_bf, W1_bf):
    mb = 128
    return pl.pallas_call(
        _w1_kernel,
        grid=(NP_PAD // mb,),
        in_specs=[
            pl.BlockSpec((mb, POOL * POOL * C), lambda s: (s, 0)),
            pl.BlockSpec((POOL * POOL * C, HIDDEN), lambda s: (0, 0)),
        ],
        out_specs=pl.BlockSpec((mb, HIDDEN), lambda s: (s, 0)),
        out_shape=jax.ShapeDtypeStruct((NP_PAD, HIDDEN), jnp.float32),
        interpret=_INTERPRET,
    )(pooled_bf, W1_bf)


def _heads_kernel(h1a_ref, b1_ref, w2_ref, b2_ref, wc_ref, bc_ref,
                  wr_ref, br_ref, scores_ref, breg_ref):
    h1 = jnp.maximum(h1a_ref[...] + b1_ref[...], 0.0)
    h2 = jax.lax.dot_general(h1, w2_ref[...], (((1,), (0,)), ((), ())),
                             precision=_PREC,
                             preferred_element_type=jnp.float32)
    h2 = jnp.maximum(h2 + b2_ref[...], 0.0)
    logit = jax.lax.dot_general(h2, wc_ref[...], (((1,), (0,)), ((), ())),
                                precision=_PREC,
                                preferred_element_type=jnp.float32)
    logit = logit + bc_ref[...]
    m = jnp.max(logit, axis=1, keepdims=True)
    e = jnp.exp(logit - m)
    scores_ref[...] = e / jnp.sum(e, axis=1, keepdims=True)
    breg = jax.lax.dot_general(h2, wr_ref[...], (((1,), (0,)), ((), ())),
                               precision=_PREC,
                               preferred_element_type=jnp.float32)
    breg_ref[...] = breg + br_ref[...]


def _binmajor_to_cmajor(pooled_bm):
    return jnp.transpose(
        pooled_bm.reshape(NP_PAD, POOL * POOL, C), (0, 2, 1)
    ).reshape(NP_PAD, C * POOL * POOL)


def _run_heads(h1a, b1, W2, b2, Wc, bc, Wr, br):
    return pl.pallas_call(
        _heads_kernel,
        out_shape=[
            jax.ShapeDtypeStruct((NP_PAD, NUM_CLASSES), jnp.float32),
            jax.ShapeDtypeStruct((NP_PAD, NUM_CLASSES * 4), jnp.float32),
        ],
        interpret=_INTERPRET,
    )(h1a, b1, W2, b2, Wc, bc, Wr, br)


NCF = NUM_CLASSES - 1
NDET_PAD = 128


def _nms_kernel(img_ref, scores_ref, deltas_ref, prop_ref,
                sel_s_ref, sx1_ref, sy1_ref, sx2_ref, sy2_ref, s_cur_ref):
    img_h = img_ref[0].astype(jnp.float32)
    img_w = img_ref[1].astype(jnp.float32)

    px1 = prop_ref[0, :]
    py1 = prop_ref[1, :]
    px2 = prop_ref[2, :]
    py2 = prop_ref[3, :]
    widths = px2 - px1
    heights = py2 - py1
    ctr_x = px1 + 0.5 * widths
    ctr_y = py1 + 0.5 * heights

    dx = deltas_ref[0, :, :] / REG_W[0]
    dy = deltas_ref[1, :, :] / REG_W[1]
    dw = jnp.minimum(deltas_ref[2, :, :] / REG_W[2], BBOX_XFORM_CLIP)
    dh = jnp.minimum(deltas_ref[3, :, :] / REG_W[3], BBOX_XFORM_CLIP)
    pcx = dx * widths + ctr_x
    pcy = dy * heights + ctr_y
    pw = jnp.exp(dw) * widths
    ph = jnp.exp(dh) * heights
    bx1 = jnp.clip(pcx - 0.5 * pw, 0.0, img_w)
    by1 = jnp.clip(pcy - 0.5 * ph, 0.0, img_h)
    bx2 = jnp.clip(pcx + 0.5 * pw, 0.0, img_w)
    by2 = jnp.clip(pcy + 0.5 * ph, 0.0, img_h)
    w = bx2 - bx1
    h = by2 - by1

    score = scores_ref[...]
    valid = (score >= SCORE_THRESH) & (w >= MIN_SIZE) & (h >= MIN_SIZE)
    neg_inf = jnp.float32(-jnp.inf)
    s_cur_ref[...] = jnp.where(valid, score, neg_inf)

    area_b = (bx2 - bx1) * (by2 - by1)
    lane = jax.lax.broadcasted_iota(jnp.int32, (NCF, NP_PAD), 1)
    det_lane = jax.lax.broadcasted_iota(jnp.int32, (NCF, NDET_PAD), 1)

    def body(i, sel_acc):
        a_s, a_x1, a_y1, a_x2, a_y2 = sel_acc
        s_cur = s_cur_ref[...]
        best_s = jnp.max(s_cur, axis=1, keepdims=True)
        eqm = s_cur == best_s
        idx = jnp.min(jnp.where(eqm, lane, NP_PAD), axis=1, keepdims=True)
        sel = lane == idx
        zero = jnp.float32(0.0)
        b_x1 = jnp.sum(jnp.where(sel, bx1, zero), axis=1, keepdims=True)
        b_y1 = jnp.sum(jnp.where(sel, by1, zero), axis=1, keepdims=True)
        b_x2 = jnp.sum(jnp.where(sel, bx2, zero), axis=1, keepdims=True)
        b_y2 = jnp.sum(jnp.where(sel, by2, zero), axis=1, keepdims=True)
        ok = jnp.isfinite(best_s)

        det_m = det_lane == i
        a_s = jnp.where(det_m, jnp.where(ok, best_s, zero), a_s)
        a_x1 = jnp.where(det_m, jnp.where(ok, b_x1, zero), a_x1)
        a_y1 = jnp.where(det_m, jnp.where(ok, b_y1, zero), a_y1)
        a_x2 = jnp.where(det_m, jnp.where(ok, b_x2, zero), a_x2)
        a_y2 = jnp.where(det_m, jnp.where(ok, b_y2, zero), a_y2)

        xx1 = jnp.maximum(b_x1, bx1)
        yy1 = jnp.maximum(b_y1, by1)
        xx2 = jnp.minimum(b_x2, bx2)
        yy2 = jnp.minimum(b_y2, by2)
        inter = jnp.clip(xx2 - xx1, 0.0) * jnp.clip(yy2 - yy1, 0.0)
        area_a = (b_x2 - b_x1) * (b_y2 - b_y1)
        ious = inter / jnp.maximum(area_a + area_b - inter, 1e-6)

        s_new = jnp.where(ious > NMS_THRESH, neg_inf, s_cur)
        s_new = jnp.where(sel, neg_inf, s_new)
        s_cur_ref[...] = s_new
        return (a_s, a_x1, a_y1, a_x2, a_y2)

    zeros = jnp.zeros((NCF, NDET_PAD), jnp.float32)
    a_s, a_x1, a_y1, a_x2, a_y2 = jax.lax.fori_loop(
        0, NUM_DET, body, (zeros, zeros, zeros, zeros, zeros))
    sel_s_ref[...] = a_s
    sx1_ref[...] = a_x1
    sy1_ref[...] = a_y1
    sx2_ref[...] = a_x2
    sy2_ref[...] = a_y2


def _run_nms(image_shape, scores_fg, deltas_fg, prop4):
    out = pl.pallas_call(
        _nms_kernel,
        in_specs=[
            pl.BlockSpec(memory_space=pltpu.SMEM),
            pl.BlockSpec((NCF, NP_PAD), lambda: (0, 0)),
            pl.BlockSpec((4, NCF, NP_PAD), lambda: (0, 0, 0)),
            pl.BlockSpec((4, NP_PAD), lambda: (0, 0)),
        ],
        out_specs=[pl.BlockSpec((NCF, NDET_PAD), lambda: (0, 0))] * 5,
        out_shape=[jax.ShapeDtypeStruct((NCF, NDET_PAD), jnp.float32)] * 5,
        scratch_shapes=[pltpu.VMEM((NCF, NP_PAD), jnp.float32)],
        interpret=_INTERPRET,
    )(image_shape, scores_fg, deltas_fg, prop4)
    return out


def kernel(feature, proposal, image_shape, W1, b1, W2, b2, Wc, bc, Wr, br):
    feat = jnp.transpose(feature[0], (1, 2, 0))
    feat_p = jnp.concatenate([feat[:1], feat, feat[-1:]], axis=0)
    feat_p = jnp.concatenate([feat_p[:, :1], feat_p, feat_p[:, -1:]], axis=1)
    feat_pad = feat_p.reshape(PADW * PADW, C)

    prop_pad = jnp.pad(proposal, ((0, NP_PAD - N_PROP), (0, 0)))
    b = prop_pad / STRIDE
    grid = (jnp.arange(POOL, dtype=jnp.float32) + 0.5) / POOL
    rx1, ry1, rx2, ry2 = b[:, 0], b[:, 1], b[:, 2], b[:, 3]
    sx = rx1[:, None] + (rx2 - rx1)[:, None] * grid[None, :] - 0.5
    sy = ry1[:, None] + (ry2 - ry1)[:, None] * grid[None, :] - 0.5
    x0f = jnp.floor(sx)
    y0f = jnp.floor(sy)
    lx = (sx - x0f).astype(jnp.float32)
    ly = (sy - y0f).astype(jnp.float32)
    qx = jnp.clip(x0f.astype(jnp.int32), -1, FW - 1) + 1
    qy = jnp.clip(y0f.astype(jnp.int32), -1, FH - 1) + 1
    base = qy[:, :, None] * PADW + qx[:, None, :]
    ind = jnp.transpose(base, (1, 2, 0)).reshape(POOL * POOL, NP_PAD)
    lyx = jnp.stack([ly.T, lx.T], axis=0)

    pooled_bm = _run_pool(ind, lyx, feat_pad)
    pooled_bf = _binmajor_to_cmajor(pooled_bm).astype(jnp.bfloat16)
    h1a = _run_w1(pooled_bf, W1.astype(jnp.bfloat16))
    scores, breg = _run_heads(h1a, b1.reshape(1, HIDDEN), W2,
                              b2.reshape(1, HIDDEN), Wc,
                              bc.reshape(1, NUM_CLASSES), Wr,
                              br.reshape(1, NUM_CLASSES * 4))

    scores_fg = scores[:, 1:].T
    scores_fg = jnp.where(
        jnp.arange(NP_PAD)[None, :] < N_PROP, scores_fg, -1.0)
    deltas = breg.reshape(NP_PAD, NUM_CLASSES, 4)[:, 1:, :]
    deltas_fg = jnp.transpose(deltas, (2, 1, 0))
    prop4 = prop_pad.T

    sel_s, sx1, sy1, sx2, sy2 = _run_nms(image_shape, scores_fg, deltas_fg,
                                         prop4)

    boxes = jnp.stack([sx1, sy1, sx2, sy2], axis=-1)
    boxes = boxes[:, :NUM_DET, :].reshape(-1, 4)
    scores_out = sel_s[:, :NUM_DET].reshape(-1)
    labels = jnp.repeat(jnp.arange(1, NUM_CLASSES, dtype=jnp.int32), NUM_DET)
    return boxes, scores_out, labels
